# baseline (speedup 1.0000x reference)
import functools

import jax
import jax.numpy as jnp
from jax import lax
from jax.experimental import pallas as pl
from jax.experimental.pallas import tpu as pltpu
from jax.experimental.pallas import tpu_sc as plsc

B, S, D = 4, 2048, 1024
_NC, _NS, _L = 2, 16, 16
_NW = _NC * _NS

_SEQ_PER_W = S // _NW
_CH_ROWS = 16
_CH = _CH_ROWS * D
_NSTEP = B * (_SEQ_PER_W // _CH_ROWS)


def _sc_body(x_hbm, emb_hbm, out_hbm, eb, xb0, xb1, xb2,
             si0, si1, si2, so0, so1, so2):
    wid = lax.axis_index("s") * _NC + lax.axis_index("c")
    sw = wid * (_SEQ_PER_W * D)
    pltpu.sync_copy(emb_hbm.at[pl.ds(sw, _SEQ_PER_W * D)], eb)

    xbufs = (xb0, xb1, xb2)
    sin = (si0, si1, si2)
    sout = (so0, so1, so2)
    nchunk = _SEQ_PER_W // _CH_ROWS

    def xoff(step):
        b, c = step // nchunk, step % nchunk
        return b * (S * D) + sw + c * _CH

    pltpu.async_copy(x_hbm.at[pl.ds(xoff(0), _CH)], xb0, si0)
    for k in range(_NSTEP):
        p = k % 3
        if k + 1 < _NSTEP:
            q = (k + 1) % 3
            if k - 2 >= 0:
                pltpu.make_async_copy(
                    xbufs[q], out_hbm.at[pl.ds(xoff(k - 2), _CH)], sout[q]
                ).wait()
            pltpu.async_copy(x_hbm.at[pl.ds(xoff(k + 1), _CH)], xbufs[q], sin[q])
        xb = xbufs[p]
        pltpu.make_async_copy(x_hbm.at[pl.ds(xoff(k), _CH)], xb, sin[p]).wait()
        ebase = (k % nchunk) * _CH

        def add8(i, _, xb=xb, ebase=ebase):
            base = i * (8 * _L)
            for u in range(8):
                o = base + u * _L
                xb[pl.ds(o, _L)] = xb[pl.ds(o, _L)] + eb[pl.ds(ebase + o, _L)]
            return 0

        lax.fori_loop(0, _CH // (8 * _L), add8, 0)
        pltpu.async_copy(xb, out_hbm.at[pl.ds(xoff(k), _CH)], sout[p])
    for k in range(_NSTEP - 3, _NSTEP):
        p = k % 3
        pltpu.make_async_copy(
            xbufs[p], out_hbm.at[pl.ds(xoff(k), _CH)], sout[p]
        ).wait()


_sc_call = functools.partial(
    pl.kernel,
    out_type=jax.ShapeDtypeStruct((B * S * D,), jnp.float32),
    mesh=plsc.VectorSubcoreMesh(core_axis_name="c", subcore_axis_name="s"),
    scratch_types=[
        pltpu.VMEM((_SEQ_PER_W * D,), jnp.float32),
        pltpu.VMEM((_CH,), jnp.float32),
        pltpu.VMEM((_CH,), jnp.float32),
        pltpu.VMEM((_CH,), jnp.float32),
        pltpu.SemaphoreType.DMA,
        pltpu.SemaphoreType.DMA,
        pltpu.SemaphoreType.DMA,
        pltpu.SemaphoreType.DMA,
        pltpu.SemaphoreType.DMA,
        pltpu.SemaphoreType.DMA,
    ],
)(_sc_body)


def kernel(x, emb_weight):
    out = _sc_call(x.reshape(-1), emb_weight.reshape(-1))
    return out.reshape(B, S, D)


# speedup vs baseline: 1.6615x; 1.6615x over previous
import functools

import jax
import jax.numpy as jnp
from jax import lax
from jax.experimental import pallas as pl
from jax.experimental.pallas import tpu as pltpu
from jax.experimental.pallas import tpu_sc as plsc

B, S, D = 4, 2048, 1024
_NC, _NS, _L = 2, 16, 16
_NW = _NC * _NS

_SEQ_PER_W = S // _NW
_CH_ROWS = 16
_NSTEP = B * (_SEQ_PER_W // _CH_ROWS)


def _sc_body(x_hbm, emb_hbm, out_hbm, eb, xb0, xb1, xb2,
             si0, si1, si2, so0, so1, so2):
    wid = lax.axis_index("s") * _NC + lax.axis_index("c")
    sw = wid * _SEQ_PER_W
    pltpu.sync_copy(emb_hbm.at[pl.ds(sw, _SEQ_PER_W)], eb)

    xbufs = (xb0, xb1, xb2)
    sin = (si0, si1, si2)
    sout = (so0, so1, so2)
    nchunk = _SEQ_PER_W // _CH_ROWS

    def xrow(step):
        b, c = step // nchunk, step % nchunk
        return b * S + sw + c * _CH_ROWS

    pltpu.async_copy(x_hbm.at[pl.ds(xrow(0), _CH_ROWS)], xb0, si0)
    for k in range(_NSTEP):
        p = k % 3
        if k + 1 < _NSTEP:
            q = (k + 1) % 3
            if k - 2 >= 0:
                pltpu.make_async_copy(
                    xbufs[q], out_hbm.at[pl.ds(xrow(k - 2), _CH_ROWS)], sout[q]
                ).wait()
            pltpu.async_copy(
                x_hbm.at[pl.ds(xrow(k + 1), _CH_ROWS)], xbufs[q], sin[q])
        xb = xbufs[p]
        pltpu.make_async_copy(
            x_hbm.at[pl.ds(xrow(k), _CH_ROWS)], xb, sin[p]).wait()
        erow = (k % nchunk) * _CH_ROWS

        def add_row(r, _, xb=xb, erow=erow):
            for u in range(D // _L):
                cs = pl.ds(u * _L, _L)
                xb[r, cs] = xb[r, cs] + eb[erow + r, cs]
            return 0

        lax.fori_loop(0, _CH_ROWS, add_row, 0)
        pltpu.async_copy(xb, out_hbm.at[pl.ds(xrow(k), _CH_ROWS)], sout[p])
    for k in range(_NSTEP - 3, _NSTEP):
        p = k % 3
        pltpu.make_async_copy(
            xbufs[p], out_hbm.at[pl.ds(xrow(k), _CH_ROWS)], sout[p]
        ).wait()


_sc_call = functools.partial(
    pl.kernel,
    out_type=jax.ShapeDtypeStruct((B * S, D), jnp.float32),
    mesh=plsc.VectorSubcoreMesh(core_axis_name="c", subcore_axis_name="s"),
    compiler_params=pltpu.CompilerParams(use_tc_tiling_on_sc=True),
    scratch_types=[
        pltpu.VMEM((_SEQ_PER_W, D), jnp.float32),
        pltpu.VMEM((_CH_ROWS, D), jnp.float32),
        pltpu.VMEM((_CH_ROWS, D), jnp.float32),
        pltpu.VMEM((_CH_ROWS, D), jnp.float32),
        pltpu.SemaphoreType.DMA,
        pltpu.SemaphoreType.DMA,
        pltpu.SemaphoreType.DMA,
        pltpu.SemaphoreType.DMA,
        pltpu.SemaphoreType.DMA,
        pltpu.SemaphoreType.DMA,
    ],
)(_sc_body)


def kernel(x, emb_weight):
    out = _sc_call(x.reshape(B * S, D), emb_weight)
    return out.reshape(B, S, D)


# speedup vs baseline: 1.8034x; 1.0854x over previous
import functools

import jax
import jax.numpy as jnp
from jax import lax
from jax.experimental import pallas as pl
from jax.experimental.pallas import tpu as pltpu
from jax.experimental.pallas import tpu_sc as plsc

B, S, D = 4, 2048, 1024
B_TC = 3
_NC, _NS, _L = 2, 16, 16
_NW = _NC * _NS

_SEQ_PER_W = S // _NW
_CH_ROWS = 16
_NCHUNK = _SEQ_PER_W // _CH_ROWS
_NSTEP = (B - B_TC) * _NCHUNK


def _sc_body(x_hbm, emb_hbm, out_hbm, eb, xb0, xb1, xb2,
             si0, si1, si2, so0, so1, so2):
    wid = lax.axis_index("s") * _NC + lax.axis_index("c")
    sw = wid * _SEQ_PER_W
    pltpu.sync_copy(emb_hbm.at[pl.ds(sw, _SEQ_PER_W)], eb)

    xbufs = (xb0, xb1, xb2)
    sin = (si0, si1, si2)
    sout = (so0, so1, so2)

    def xrow(step):
        b, c = step // _NCHUNK, step % _NCHUNK
        return (B_TC + b) * S + sw + c * _CH_ROWS

    def orow(step):
        b, c = step // _NCHUNK, step % _NCHUNK
        return b * S + sw + c * _CH_ROWS

    pltpu.async_copy(x_hbm.at[pl.ds(xrow(0), _CH_ROWS)], xb0, si0)
    for k in range(_NSTEP):
        p = k % 3
        if k + 1 < _NSTEP:
            q = (k + 1) % 3
            if k - 2 >= 0:
                pltpu.make_async_copy(
                    xbufs[q], out_hbm.at[pl.ds(orow(k - 2), _CH_ROWS)], sout[q]
                ).wait()
            pltpu.async_copy(
                x_hbm.at[pl.ds(xrow(k + 1), _CH_ROWS)], xbufs[q], sin[q])
        xb = xbufs[p]
        pltpu.make_async_copy(
            x_hbm.at[pl.ds(xrow(k), _CH_ROWS)], xb, sin[p]).wait()
        erow = (k % _NCHUNK) * _CH_ROWS

        def add_row(r, _, xb=xb, erow=erow):
            for u in range(D // _L):
                cs = pl.ds(u * _L, _L)
                xb[r, cs] = xb[r, cs] + eb[erow + r, cs]
            return 0

        lax.fori_loop(0, _CH_ROWS, add_row, 0)
        pltpu.async_copy(xb, out_hbm.at[pl.ds(orow(k), _CH_ROWS)], sout[p])
    for k in range(max(0, _NSTEP - 3), _NSTEP):
        p = k % 3
        pltpu.make_async_copy(
            xbufs[p], out_hbm.at[pl.ds(orow(k), _CH_ROWS)], sout[p]
        ).wait()


_sc_call = functools.partial(
    pl.kernel,
    out_type=jax.ShapeDtypeStruct(((B - B_TC) * S, D), jnp.float32),
    mesh=plsc.VectorSubcoreMesh(core_axis_name="c", subcore_axis_name="s"),
    compiler_params=pltpu.CompilerParams(use_tc_tiling_on_sc=True),
    scratch_types=[
        pltpu.VMEM((_SEQ_PER_W, D), jnp.float32),
        pltpu.VMEM((_CH_ROWS, D), jnp.float32),
        pltpu.VMEM((_CH_ROWS, D), jnp.float32),
        pltpu.VMEM((_CH_ROWS, D), jnp.float32),
        pltpu.SemaphoreType.DMA,
        pltpu.SemaphoreType.DMA,
        pltpu.SemaphoreType.DMA,
        pltpu.SemaphoreType.DMA,
        pltpu.SemaphoreType.DMA,
        pltpu.SemaphoreType.DMA,
    ],
)(_sc_body)


def _tc_body(x_ref, emb_ref, o_ref):
    o_ref[...] = x_ref[...] + emb_ref[...]


def _tc_call(x, emb_weight):
    return pl.pallas_call(
        _tc_body,
        grid=(B_TC,),
        in_specs=[
            pl.BlockSpec((1, S, D), lambda b: (b, 0, 0)),
            pl.BlockSpec((S, D), lambda b: (0, 0)),
        ],
        out_specs=pl.BlockSpec((1, S, D), lambda b: (b, 0, 0)),
        out_shape=jax.ShapeDtypeStruct((B_TC, S, D), x.dtype),
    )(x, emb_weight)


def kernel(x, emb_weight):
    sc_out = _sc_call(x.reshape(B * S, D), emb_weight)
    tc_out = _tc_call(x, emb_weight)
    return jnp.concatenate(
        [tc_out, sc_out.reshape(B - B_TC, S, D)], axis=0)


# speedup vs baseline: 4.7430x; 2.6301x over previous
import jax
import jax.numpy as jnp
from jax.experimental import pallas as pl
from jax.experimental.pallas import tpu as pltpu

B, S, D = 4, 2048, 1024
R = 512
NCH = S // R
NSTEP = NCH * B


def _body(x_hbm, emb_hbm, o_hbm, eb0, eb1, xb0, xb1, xb2,
          se0, se1, si0, si1, si2, so0, so1, so2):
    ebufs, se = (eb0, eb1), (se0, se1)
    xbufs, si, so = (xb0, xb1, xb2), (si0, si1, si2), (so0, so1, so2)

    def bc(k):
        return k % B, k // B

    def ecopy(c):
        return pltpu.make_async_copy(
            emb_hbm.at[pl.ds(c * R, R)], ebufs[c % 2], se[c % 2])

    def xcopy(k):
        b, c = bc(k)
        return pltpu.make_async_copy(
            x_hbm.at[b, pl.ds(c * R, R)], xbufs[k % 3], si[k % 3])

    def ocopy(k):
        b, c = bc(k)
        return pltpu.make_async_copy(
            xbufs[k % 3], o_hbm.at[b, pl.ds(c * R, R)], so[k % 3])

    ecopy(0).start()
    xcopy(0).start()
    for k in range(NSTEP):
        b, c = bc(k)
        if k + 1 < NSTEP:
            if k >= 2:
                ocopy(k - 2).wait()
            xcopy(k + 1).start()
        if b == 0 and c + 1 < NCH:
            ecopy(c + 1).start()
        xcopy(k).wait()
        if b == 0:
            ecopy(c).wait()
        xb = xbufs[k % 3]
        xb[...] = xb[...] + ebufs[c % 2][...]
        ocopy(k).start()
    for k in range(NSTEP - 3, NSTEP):
        ocopy(k).wait()


def kernel(x, emb_weight):
    return pl.pallas_call(
        _body,
        in_specs=[
            pl.BlockSpec(memory_space=pl.ANY),
            pl.BlockSpec(memory_space=pl.ANY),
        ],
        out_specs=pl.BlockSpec(memory_space=pl.ANY),
        out_shape=jax.ShapeDtypeStruct(x.shape, x.dtype),
        scratch_shapes=[
            pltpu.VMEM((R, D), jnp.float32),
            pltpu.VMEM((R, D), jnp.float32),
            pltpu.VMEM((R, D), jnp.float32),
            pltpu.VMEM((R, D), jnp.float32),
            pltpu.VMEM((R, D), jnp.float32),
            pltpu.SemaphoreType.DMA,
            pltpu.SemaphoreType.DMA,
            pltpu.SemaphoreType.DMA,
            pltpu.SemaphoreType.DMA,
            pltpu.SemaphoreType.DMA,
            pltpu.SemaphoreType.DMA,
            pltpu.SemaphoreType.DMA,
            pltpu.SemaphoreType.DMA,
        ],
    )(x, emb_weight)


# speedup vs baseline: 5.0991x; 1.0751x over previous
import jax
import jax.numpy as jnp
from jax.experimental import pallas as pl
from jax.experimental.pallas import tpu as pltpu

B, S, D = 4, 2048, 1024
R = 1024
NCH = S // R
NSTEP = NCH * B


def _body(x_hbm, emb_hbm, o_hbm, eb0, eb1, xb0, xb1, xb2,
          se0, se1, si0, si1, si2, so0, so1, so2):
    ebufs, se = (eb0, eb1), (se0, se1)
    xbufs, si, so = (xb0, xb1, xb2), (si0, si1, si2), (so0, so1, so2)

    def bc(k):
        return k % B, k // B

    def ecopy(c):
        return pltpu.make_async_copy(
            emb_hbm.at[pl.ds(c * R, R)], ebufs[c % 2], se[c % 2])

    def xcopy(k):
        b, c = bc(k)
        return pltpu.make_async_copy(
            x_hbm.at[b, pl.ds(c * R, R)], xbufs[k % 3], si[k % 3])

    def ocopy(k):
        b, c = bc(k)
        return pltpu.make_async_copy(
            xbufs[k % 3], o_hbm.at[b, pl.ds(c * R, R)], so[k % 3])

    ecopy(0).start()
    xcopy(0).start()
    for k in range(NSTEP):
        b, c = bc(k)
        if k + 1 < NSTEP:
            if k >= 2:
                ocopy(k - 2).wait()
            xcopy(k + 1).start()
        if b == 0 and c + 1 < NCH:
            ecopy(c + 1).start()
        xcopy(k).wait()
        if b == 0:
            ecopy(c).wait()
        xb = xbufs[k % 3]
        xb[...] = xb[...] + ebufs[c % 2][...]
        ocopy(k).start()
    for k in range(NSTEP - 3, NSTEP):
        ocopy(k).wait()


def kernel(x, emb_weight):
    return pl.pallas_call(
        _body,
        in_specs=[
            pl.BlockSpec(memory_space=pl.ANY),
            pl.BlockSpec(memory_space=pl.ANY),
        ],
        out_specs=pl.BlockSpec(memory_space=pl.ANY),
        out_shape=jax.ShapeDtypeStruct(x.shape, x.dtype),
        scratch_shapes=[
            pltpu.VMEM((R, D), jnp.float32),
            pltpu.VMEM((R, D), jnp.float32),
            pltpu.VMEM((R, D), jnp.float32),
            pltpu.VMEM((R, D), jnp.float32),
            pltpu.VMEM((R, D), jnp.float32),
            pltpu.SemaphoreType.DMA,
            pltpu.SemaphoreType.DMA,
            pltpu.SemaphoreType.DMA,
            pltpu.SemaphoreType.DMA,
            pltpu.SemaphoreType.DMA,
            pltpu.SemaphoreType.DMA,
            pltpu.SemaphoreType.DMA,
            pltpu.SemaphoreType.DMA,
        ],
    )(x, emb_weight)


# speedup vs baseline: 5.3394x; 1.0471x over previous
import jax
import jax.numpy as jnp
from jax.experimental import pallas as pl
from jax.experimental.pallas import tpu as pltpu

B, S, D = 4, 2048, 1024
R = 2048
NCH = S // R
NSTEP = NCH * B


def _body(x_hbm, emb_hbm, o_hbm, eb0, eb1, xb0, xb1, xb2,
          se0, se1, si0, si1, si2, so0, so1, so2):
    ebufs, se = (eb0, eb1), (se0, se1)
    xbufs, si, so = (xb0, xb1, xb2), (si0, si1, si2), (so0, so1, so2)

    def bc(k):
        return k % B, k // B

    def ecopy(c):
        return pltpu.make_async_copy(
            emb_hbm.at[pl.ds(c * R, R)], ebufs[c % 2], se[c % 2])

    def xcopy(k):
        b, c = bc(k)
        return pltpu.make_async_copy(
            x_hbm.at[b, pl.ds(c * R, R)], xbufs[k % 3], si[k % 3])

    def ocopy(k):
        b, c = bc(k)
        return pltpu.make_async_copy(
            xbufs[k % 3], o_hbm.at[b, pl.ds(c * R, R)], so[k % 3])

    ecopy(0).start()
    xcopy(0).start()
    for k in range(NSTEP):
        b, c = bc(k)
        if k + 1 < NSTEP:
            if k >= 2:
                ocopy(k - 2).wait()
            xcopy(k + 1).start()
        if b == 0 and c + 1 < NCH:
            ecopy(c + 1).start()
        xcopy(k).wait()
        if b == 0:
            ecopy(c).wait()
        xb = xbufs[k % 3]
        xb[...] = xb[...] + ebufs[c % 2][...]
        ocopy(k).start()
    for k in range(NSTEP - 3, NSTEP):
        ocopy(k).wait()


def kernel(x, emb_weight):
    return pl.pallas_call(
        _body,
        in_specs=[
            pl.BlockSpec(memory_space=pl.ANY),
            pl.BlockSpec(memory_space=pl.ANY),
        ],
        out_specs=pl.BlockSpec(memory_space=pl.ANY),
        out_shape=jax.ShapeDtypeStruct(x.shape, x.dtype),
        scratch_shapes=[
            pltpu.VMEM((R, D), jnp.float32),
            pltpu.VMEM((R, D), jnp.float32),
            pltpu.VMEM((R, D), jnp.float32),
            pltpu.VMEM((R, D), jnp.float32),
            pltpu.VMEM((R, D), jnp.float32),
            pltpu.SemaphoreType.DMA,
            pltpu.SemaphoreType.DMA,
            pltpu.SemaphoreType.DMA,
            pltpu.SemaphoreType.DMA,
            pltpu.SemaphoreType.DMA,
            pltpu.SemaphoreType.DMA,
            pltpu.SemaphoreType.DMA,
            pltpu.SemaphoreType.DMA,
        ],
    )(x, emb_weight)


# speedup vs baseline: 5.4471x; 1.0202x over previous
import jax
import jax.numpy as jnp
from jax.experimental import pallas as pl
from jax.experimental.pallas import tpu as pltpu

B, S, D = 4, 2048, 1024


def _body(x_hbm, emb_hbm, o_hbm, eb, xb0, xb1, xb2, xb3,
          se, si0, si1, si2, si3, so0, so1, so2, so3):
    xbufs = (xb0, xb1, xb2, xb3)
    si = (si0, si1, si2, si3)
    so = (so0, so1, so2, so3)

    def xcopy(b):
        return pltpu.make_async_copy(x_hbm.at[b], xbufs[b], si[b])

    def ocopy(b):
        return pltpu.make_async_copy(xbufs[b], o_hbm.at[b], so[b])

    ecopy = pltpu.make_async_copy(emb_hbm, eb, se)
    ecopy.start()
    for b in range(B):
        xcopy(b).start()
    ecopy.wait()
    for b in range(B):
        xcopy(b).wait()
        xb = xbufs[b]
        xb[...] = xb[...] + eb[...]
        ocopy(b).start()
    for b in range(B):
        ocopy(b).wait()


def kernel(x, emb_weight):
    return pl.pallas_call(
        _body,
        in_specs=[
            pl.BlockSpec(memory_space=pl.ANY),
            pl.BlockSpec(memory_space=pl.ANY),
        ],
        out_specs=pl.BlockSpec(memory_space=pl.ANY),
        out_shape=jax.ShapeDtypeStruct(x.shape, x.dtype),
        scratch_shapes=[
            pltpu.VMEM((S, D), jnp.float32),
            pltpu.VMEM((S, D), jnp.float32),
            pltpu.VMEM((S, D), jnp.float32),
            pltpu.VMEM((S, D), jnp.float32),
            pltpu.VMEM((S, D), jnp.float32),
            pltpu.SemaphoreType.DMA,
            pltpu.SemaphoreType.DMA,
            pltpu.SemaphoreType.DMA,
            pltpu.SemaphoreType.DMA,
            pltpu.SemaphoreType.DMA,
            pltpu.SemaphoreType.DMA,
            pltpu.SemaphoreType.DMA,
            pltpu.SemaphoreType.DMA,
            pltpu.SemaphoreType.DMA,
        ],
    )(x, emb_weight)
